# Initial kernel scaffold; baseline (speedup 1.0000x reference)
import jax
import jax.numpy as jnp
from jax.experimental import pallas as pl


def kernel(logits, labels):
    raise NotImplementedError("write your pallas kernel here")


# speedup vs baseline: 16.8989x; 16.8989x over previous
import functools

import jax
import jax.numpy as jnp
from jax import lax
from jax.experimental import pallas as pl
from jax.experimental.pallas import tpu as pltpu
from jax.experimental.pallas import tpu_sc as plsc

B = 8
N = 512 * 512
NC, NS, L = 2, 16, 16
NW = NC * NS
WPI = NW // B
PER_W = N // WPI
CH = 8192
NCH = PER_W // CH
K = 2048
EMAX = 8.0
SCALE = K / EMAX
PACK = 13


def _sc_body(lg_hbm, lb_hbm, cp_out, fs_out, pt_out, lbuf, ybuf, hcp, hf, ptb):
    c = lax.axis_index("c")
    s = lax.axis_index("s")
    wid = c * NS + s
    img = wid // WPI
    base = img * N + (wid % WPI) * PER_W
    lanes = lax.iota(jnp.int32, L)

    def zbody(i, carry):
        hcp[pl.ds(i * L, L)] = jnp.zeros((L,), jnp.int32)
        hf[pl.ds(i * L, L)] = jnp.zeros((L,), jnp.float32)
        return carry
    lax.fori_loop(0, L * K // L, zbody, 0)
    lanebase = lanes * K

    def chunk_body(ci, acc):
        start = base + ci * CH
        pltpu.sync_copy(lg_hbm.at[pl.ds(start, CH)], lbuf)
        pltpu.sync_copy(lb_hbm.at[pl.ds(start, CH)], ybuf)

        def vbody(v, a):
            lv = lbuf[pl.ds(v * L, L)]
            y = ybuf[pl.ds(v * L, L)]
            yf = y.astype(jnp.float32)
            e = 1.0 - lv * (2.0 * yf - 1.0)
            m = e > 0.0
            b = jnp.clip(((EMAX - e) * SCALE).astype(jnp.int32), 0, K - 1)
            idx = lanebase + b
            plsc.addupdate_scatter(hcp, [idx], 1 + (y << PACK), mask=m)
            plsc.addupdate_scatter(hf, [idx], e, mask=m)
            return a + yf
        return lax.fori_loop(0, CH // L, vbody, acc)

    acc = lax.fori_loop(0, NCH, chunk_body, jnp.zeros((L,), jnp.float32))
    ptb[...] = acc
    pltpu.sync_copy(hcp, cp_out.at[wid])
    pltpu.sync_copy(hf, fs_out.at[wid])
    pltpu.sync_copy(ptb, pt_out.at[wid])


def _sc_hist(lg_flat, lb_flat):
    mesh = plsc.VectorSubcoreMesh(
        core_axis_name="c", subcore_axis_name="s", num_cores=NC, num_subcores=NS)
    f = functools.partial(
        pl.kernel,
        out_type=(
            jax.ShapeDtypeStruct((NW, L * K), jnp.int32),
            jax.ShapeDtypeStruct((NW, L * K), jnp.float32),
            jax.ShapeDtypeStruct((NW, L), jnp.float32),
        ),
        mesh=mesh,
        compiler_params=pltpu.CompilerParams(needs_layout_passes=False),
        scratch_types=[
            pltpu.VMEM((CH,), jnp.float32),
            pltpu.VMEM((CH,), jnp.int32),
            pltpu.VMEM((L * K,), jnp.int32),
            pltpu.VMEM((L * K,), jnp.float32),
            pltpu.VMEM((L,), jnp.float32),
        ],
    )(_sc_body)
    return f(lg_flat, lb_flat)


def _epi_body(cp_ref, fs_ref, pt_ref, out_ref):
    cp = cp_ref[...]
    fs = fs_ref[...]
    pt = pt_ref[...]
    cnt = jnp.bitwise_and(cp, (1 << PACK) - 1).astype(jnp.float32)
    pos = jnp.right_shift(cp, PACK).astype(jnp.float32)

    rows = NW * L
    ri = lax.broadcasted_iota(jnp.int32, (B, rows), 1)
    ii = lax.broadcasted_iota(jnp.int32, (B, rows), 0)
    S = (ri // (rows // B) == ii).astype(jnp.float32)
    cnt8 = jnp.dot(S, cnt, preferred_element_type=jnp.float32)
    pos8 = jnp.dot(S, pos, preferred_element_type=jnp.float32)
    fs8 = jnp.dot(S, fs, preferred_element_type=jnp.float32)

    wi = lax.broadcasted_iota(jnp.int32, (B, NW), 1)
    i2 = lax.broadcasted_iota(jnp.int32, (B, NW), 0)
    S2 = (wi // WPI == i2).astype(jnp.float32)
    P8 = jnp.sum(jnp.dot(S2, pt, preferred_element_type=jnp.float32),
                 axis=1, keepdims=True)

    bi = lax.broadcasted_iota(jnp.int32, (K, K), 0)
    bj = lax.broadcasted_iota(jnp.int32, (K, K), 1)
    LT = (bi <= bj).astype(jnp.float32)
    Nend = jnp.dot(cnt8, LT, preferred_element_type=jnp.float32)
    Cend = jnp.dot(pos8, LT, preferred_element_type=jnp.float32)

    U = P8 + Nend - Cend
    J = jnp.where(U > 0.0, 1.0 - (P8 - Cend) / jnp.where(U > 0.0, U, 1.0), 0.0)
    Jprev = jnp.concatenate([jnp.zeros((B, 1), jnp.float32), J[:, :-1]], axis=1)
    g = fs8 / jnp.maximum(cnt8, 1.0)
    li = jnp.sum(g * (J - Jprev), axis=1)
    out_ref[0, 0] = jnp.sum(li) * (1.0 / B)


def _tc_epilogue(cp, fs, pt):
    return pl.pallas_call(
        _epi_body,
        out_shape=jax.ShapeDtypeStruct((1, 1), jnp.float32),
        out_specs=pl.BlockSpec(memory_space=pltpu.SMEM),
    )(cp, fs, pt)


def kernel(logits, labels):
    lg = logits.reshape(B * N)
    lb = labels.reshape(B * N).astype(jnp.int32)
    cp, fs, pt = _sc_hist(lg, lb)
    out = _tc_epilogue(cp.reshape(NW * L, K), fs.reshape(NW * L, K), pt)
    return out[0, 0]


# speedup vs baseline: 36.0908x; 2.1357x over previous
import functools

import jax
import jax.numpy as jnp
from jax import lax
from jax.experimental import pallas as pl
from jax.experimental.pallas import tpu as pltpu
from jax.experimental.pallas import tpu_sc as plsc

B = 8
N = 512 * 512
NC, NS, L = 2, 16, 16
NW = NC * NS
WPI = NW // B
PER_W = N // WPI
CH = 8192
NCH = PER_W // CH
K = 2048
EMAX = 8.0
SCALE = K / EMAX
PACK = 13
UNROLL = 8


def _sc_body(lg_hbm, lb_hbm, cnt_out, pos_out, fs_out, pt_out,
             lbuf, ybuf, hcp, hf, mcnt, mpos, mf, ptb, lsem, ysem):
    c = lax.axis_index("c")
    s = lax.axis_index("s")
    wid = c * NS + s
    img = wid // WPI
    base = img * N + (wid % WPI) * PER_W
    lanes = lax.iota(jnp.int32, L)
    lanebase = lanes * K

    def zbody(i, carry):
        hcp[pl.ds(i * L, L)] = jnp.zeros((L,), jnp.int32)
        hf[pl.ds(i * L, L)] = jnp.zeros((L,), jnp.float32)
        return carry
    lax.fori_loop(0, K, zbody, 0)

    def start(ci, slot):
        pltpu.async_copy(lg_hbm.at[pl.ds(base + ci * CH, CH)],
                         lbuf.at[slot], lsem.at[slot])
        pltpu.async_copy(lb_hbm.at[pl.ds(base + ci * CH, CH)],
                         ybuf.at[slot], ysem.at[slot])

    def wait(slot):
        pltpu.make_async_copy(lg_hbm.at[pl.ds(0, CH)], lbuf.at[slot],
                              lsem.at[slot]).wait()
        pltpu.make_async_copy(lb_hbm.at[pl.ds(0, CH)], ybuf.at[slot],
                              ysem.at[slot]).wait()

    start(0, 0)

    def chunk_body(ci, acc):
        slot = lax.rem(ci, 2)
        start(lax.rem(ci + 1, NCH), lax.rem(ci + 1, 2))
        wait(slot)

        def vbody(v, a):
            offs = [v * (L * UNROLL) + u * L for u in range(UNROLL)]
            lvs = [lbuf[slot, pl.ds(o, L)] for o in offs]
            ys = [ybuf[slot, pl.ds(o, L)] for o in offs]
            mys = [y != 0 for y in ys]
            es = [jnp.where(my, 1.0 - lv, 1.0 + lv)
                  for my, lv in zip(mys, lvs)]
            ms = [e > 0.0 for e in es]
            ts = [jnp.minimum(jnp.maximum(e * (-SCALE) + float(K), 0.0),
                              float(K - 1)) for e in es]
            idxs = [lanebase + t.astype(jnp.int32) for t in ts]
            vals = [jnp.where(my, (1 << PACK) + 1, 1) for my in mys]
            for u in range(UNROLL):
                plsc.addupdate_scatter(hcp, [idxs[u]], vals[u], mask=ms[u])
                plsc.addupdate_scatter(hf, [idxs[u]], es[u], mask=ms[u])
            for y in ys:
                a = a + y
            return a
        return lax.fori_loop(0, CH // (L * UNROLL), vbody, acc)

    acc = lax.fori_loop(0, NCH, chunk_body, jnp.zeros((L,), jnp.int32))
    wait(0)
    ptb[...] = acc

    def mbody(j, carry):
        col = j * L
        v = hcp[pl.ds(col, L)]
        a_cnt = v & ((1 << PACK) - 1)
        a_pos = v >> PACK
        a_f = hf[pl.ds(col, L)]
        for r in range(1, L):
            v = hcp[pl.ds(r * K + col, L)]
            a_cnt = a_cnt + (v & ((1 << PACK) - 1))
            a_pos = a_pos + (v >> PACK)
            a_f = a_f + hf[pl.ds(r * K + col, L)]
        mcnt[pl.ds(col, L)] = a_cnt
        mpos[pl.ds(col, L)] = a_pos
        mf[pl.ds(col, L)] = a_f
        return carry
    lax.fori_loop(0, K // L, mbody, 0)

    pltpu.sync_copy(mcnt, cnt_out.at[wid])
    pltpu.sync_copy(mpos, pos_out.at[wid])
    pltpu.sync_copy(mf, fs_out.at[wid])
    pltpu.sync_copy(ptb, pt_out.at[wid])


def _sc_hist(lg_flat, lb_flat):
    mesh = plsc.VectorSubcoreMesh(
        core_axis_name="c", subcore_axis_name="s", num_cores=NC, num_subcores=NS)
    f = functools.partial(
        pl.kernel,
        out_type=(
            jax.ShapeDtypeStruct((NW, K), jnp.int32),
            jax.ShapeDtypeStruct((NW, K), jnp.int32),
            jax.ShapeDtypeStruct((NW, K), jnp.float32),
            jax.ShapeDtypeStruct((NW, L), jnp.int32),
        ),
        mesh=mesh,
        compiler_params=pltpu.CompilerParams(needs_layout_passes=False),
        scratch_types=[
            pltpu.VMEM((2, CH), jnp.float32),
            pltpu.VMEM((2, CH), jnp.int32),
            pltpu.VMEM((L * K,), jnp.int32),
            pltpu.VMEM((L * K,), jnp.float32),
            pltpu.VMEM((K,), jnp.int32),
            pltpu.VMEM((K,), jnp.int32),
            pltpu.VMEM((K,), jnp.float32),
            pltpu.VMEM((L,), jnp.int32),
            pltpu.SemaphoreType.DMA((2,)),
            pltpu.SemaphoreType.DMA((2,)),
        ],
    )(_sc_body)
    return f(lg_flat, lb_flat)


def _epi_body(cnt_ref, pos_ref, fs_ref, pt_ref, out_ref):
    cnt = cnt_ref[...].astype(jnp.float32)
    pos = pos_ref[...].astype(jnp.float32)
    fs = fs_ref[...]
    pt = pt_ref[...].astype(jnp.float32)

    wi = lax.broadcasted_iota(jnp.int32, (B, NW), 1)
    ii = lax.broadcasted_iota(jnp.int32, (B, NW), 0)
    S = (wi // WPI == ii).astype(jnp.float32)
    cnt8 = jnp.dot(S, cnt, preferred_element_type=jnp.float32)
    pos8 = jnp.dot(S, pos, preferred_element_type=jnp.float32)
    fs8 = jnp.dot(S, fs, preferred_element_type=jnp.float32)
    P8 = jnp.sum(jnp.dot(S, pt, preferred_element_type=jnp.float32),
                 axis=1, keepdims=True)

    bi = lax.broadcasted_iota(jnp.int32, (K, K), 0)
    bj = lax.broadcasted_iota(jnp.int32, (K, K), 1)
    LT = (bi <= bj).astype(jnp.float32)
    Nend = jnp.dot(cnt8, LT, preferred_element_type=jnp.float32)
    Cend = jnp.dot(pos8, LT, preferred_element_type=jnp.float32)

    U = P8 + Nend - Cend
    J = jnp.where(U > 0.0, 1.0 - (P8 - Cend) / jnp.where(U > 0.0, U, 1.0), 0.0)
    Jprev = jnp.concatenate([jnp.zeros((B, 1), jnp.float32), J[:, :-1]], axis=1)
    g = fs8 / jnp.maximum(cnt8, 1.0)
    li = jnp.sum(g * (J - Jprev), axis=1)
    out_ref[0, 0] = jnp.sum(li) * (1.0 / B)


def _tc_epilogue(cnt, pos, fs, pt):
    return pl.pallas_call(
        _epi_body,
        out_shape=jax.ShapeDtypeStruct((1, 1), jnp.float32),
        out_specs=pl.BlockSpec(memory_space=pltpu.SMEM),
    )(cnt, pos, fs, pt)


def kernel(logits, labels):
    lg = logits.reshape(B * N)
    lb = labels.reshape(B * N).astype(jnp.int32)
    cnt, pos, fs, pt = _sc_hist(lg, lb)
    out = _tc_epilogue(cnt, pos, fs, pt)
    return out[0, 0]
